# Initial kernel scaffold; baseline (speedup 1.0000x reference)
import jax
import jax.numpy as jnp
from jax.experimental import pallas as pl


def kernel(x, edge_index, gamma, beta, W0, b0, Wt, bt, W1, b1, W2, b2, Wc, bc, We, be):
    raise NotImplementedError("write your pallas kernel here")


# speedup vs baseline: 6.1350x; 6.1350x over previous
import functools

import jax
import jax.numpy as jnp
from jax import lax
from jax.experimental import pallas as pl
from jax.experimental.pallas import tpu as pltpu
from jax.experimental.pallas import tpu_sc as plsc

N = 10000
D = 128
E = 640000
K = 6

NW = 32
NS = 16
CHUNK = 128
IB = 16
OUTER = 10
CPT = IB * OUTER
EPT = CPT * CHUNK
E_PAD = NW * EPT
N_PAD = 10016
RPT = 624
ZREM = N_PAD - NS * RPT
OREM = N - NS * RPT

_mesh = plsc.VectorSubcoreMesh(core_axis_name="c", subcore_axis_name="s")


def _sc_deg_body(col_hbm, out_hbm, colv, degv, sem):
    cid = lax.axis_index("c")
    sid = lax.axis_index("s")
    wid = cid * NS + sid
    pltpu.async_copy(col_hbm.at[wid], colv, sem).wait()
    zeros16 = jnp.zeros((16,), jnp.float32)

    def zbody(i, carry):
        degv[pl.ds(i * 16, 16)] = zeros16
        return carry

    lax.fori_loop(0, N_PAD // 16, zbody, 0)
    ones16 = jnp.ones((16,), jnp.float32)

    def ebody(j, carry):
        for c in range(CHUNK // 16):
            idx = colv[j, pl.ds(c * 16, 16)]
            plsc.addupdate_scatter(degv, [idx], ones16)
        return carry

    lax.fori_loop(0, CPT, ebody, 0)
    pltpu.sync_copy(degv, out_hbm.at[wid])


_sc_deg = pl.kernel(
    _sc_deg_body,
    out_type=jax.ShapeDtypeStruct((NW, N_PAD), jnp.float32),
    mesh=_mesh,
    compiler_params=pltpu.CompilerParams(needs_layout_passes=False),
    scratch_types=[
        pltpu.VMEM((CPT, CHUNK), jnp.int32),
        pltpu.VMEM((N_PAD,), jnp.float32),
        pltpu.SemaphoreType.DMA,
    ],
)


def _sc_hop_body(g_hbm, row_hbm, col_hbm, zer_hbm, out_hbm, rowb, colb, rows, acc, sem):
    cid = lax.axis_index("c")
    sid = lax.axis_index("s")
    wid = cid * NS + sid
    pltpu.sync_copy(zer_hbm.at[pl.ds(sid * RPT, RPT)], acc.at[pl.ds(sid * RPT, RPT)])

    @pl.when(sid == 0)
    def _():
        pltpu.sync_copy(zer_hbm.at[pl.ds(NS * RPT, ZREM)], acc.at[pl.ds(NS * RPT, ZREM)])

    plsc.subcore_barrier()

    def obody(o, carry):
        pltpu.sync_copy(row_hbm.at[wid, pl.ds(o * IB, IB)], rowb)
        pltpu.sync_copy(col_hbm.at[wid, pl.ds(o * IB, IB)], colb)

        def ibody(j, c2):
            pltpu.async_copy(g_hbm.at[rowb.at[j]], rows, sem).wait()
            pltpu.sync_copy(rows, acc.at[colb.at[j]], add=True)
            return c2

        lax.fori_loop(0, IB, ibody, 0)
        return carry

    lax.fori_loop(0, OUTER, obody, 0)
    plsc.subcore_barrier()
    pltpu.sync_copy(acc.at[pl.ds(sid * RPT, RPT)], out_hbm.at[cid, pl.ds(sid * RPT, RPT)])

    @pl.when(sid == 0)
    def _():
        pltpu.sync_copy(acc.at[pl.ds(NS * RPT, OREM)], out_hbm.at[cid, pl.ds(NS * RPT, OREM)])


_sc_hop = pl.kernel(
    _sc_hop_body,
    out_type=jax.ShapeDtypeStruct((2, N, D), jnp.float32),
    mesh=_mesh,
    compiler_params=pltpu.CompilerParams(needs_layout_passes=False),
    scratch_types=[
        pltpu.VMEM((IB, CHUNK), jnp.int32),
        pltpu.VMEM((IB, CHUNK), jnp.int32),
        pltpu.VMEM((CHUNK, D), jnp.float32),
        pltpu.VMEM_SHARED((N_PAD, D), jnp.float32),
        pltpu.SemaphoreType.DMA,
    ],
)


def _gelu(x):
    return x * 0.5 * (1.0 + lax.erf(x * 0.7071067811865476))


def _tc_pre_body(x_ref, degp_ref, gamma_ref, beta_ref, w0_ref, b0_ref, wt0_ref,
                 xb_ref, g_ref, acc_ref, dis_ref):
    x = x_ref[...]
    mu = jnp.mean(x, axis=0, keepdims=True)
    xc = x - mu
    var = jnp.mean(xc * xc, axis=0, keepdims=True)
    xb = xc * lax.rsqrt(var + 1e-5) * gamma_ref[...] + beta_ref[...]
    xb_ref[...] = xb
    h = _gelu(jnp.dot(xb, w0_ref[...], preferred_element_type=jnp.float32) + b0_ref[...])
    deg = jnp.sum(degp_ref[...], axis=1, keepdims=True)[:N]
    dis = jnp.where(deg > 0, lax.rsqrt(jnp.maximum(deg, 1e-12)), 0.0)
    dis_ref[...] = dis
    g_ref[...] = dis * h
    acc_ref[...] = jnp.dot(h, wt0_ref[...], preferred_element_type=jnp.float32)


_tc_pre = pl.pallas_call(
    _tc_pre_body,
    out_shape=(
        jax.ShapeDtypeStruct((N, D), jnp.float32),
        jax.ShapeDtypeStruct((N, D), jnp.float32),
        jax.ShapeDtypeStruct((N, D), jnp.float32),
        jax.ShapeDtypeStruct((N, 1), jnp.float32),
    ),
)


def _tc_hop_body(s_ref, dis_ref, wt_ref, acc_in_ref, g_ref, acc_out_ref):
    s = s_ref[0] + s_ref[1]
    dis = dis_ref[...]
    cur = dis * s
    g_ref[...] = dis * cur
    acc_out_ref[...] = acc_in_ref[...] + jnp.dot(
        cur, wt_ref[...], preferred_element_type=jnp.float32)


_tc_hop = pl.pallas_call(
    _tc_hop_body,
    out_shape=(
        jax.ShapeDtypeStruct((N, D), jnp.float32),
        jax.ShapeDtypeStruct((N, D), jnp.float32),
    ),
)


def _tc_post_body(xb_ref, acc_ref, bt_ref, w1a_ref, w1b_ref, b1_ref,
                  w2_ref, b2_ref, wc_ref, bc_ref, we_ref, be_ref,
                  a_ref, cl_ref):
    h = _gelu(acc_ref[...] + bt_ref[...])
    a1 = _gelu(jnp.dot(xb_ref[...], w1a_ref[...], preferred_element_type=jnp.float32)
               + jnp.dot(h, w1b_ref[...], preferred_element_type=jnp.float32)
               + b1_ref[...])
    a2 = _gelu(jnp.dot(a1, w2_ref[...], preferred_element_type=jnp.float32) + b2_ref[...])
    a_ref[...] = a2
    cl = _gelu(jnp.dot(a2, wc_ref[...], preferred_element_type=jnp.float32) + bc_ref[...])
    logits = jnp.dot(cl, we_ref[...], preferred_element_type=jnp.float32) + be_ref[...]
    m = jnp.max(logits, axis=1, keepdims=True)
    e = jnp.exp(logits - m)
    cl_ref[...] = e / jnp.sum(e, axis=1, keepdims=True)


_tc_post = pl.pallas_call(
    _tc_post_body,
    out_shape=(
        jax.ShapeDtypeStruct((N, 64), jnp.float32),
        jax.ShapeDtypeStruct((N, 16), jnp.float32),
    ),
)


def kernel(x, edge_index, gamma, beta, W0, b0, Wt, bt, W1, b1, W2, b2, Wc, bc, We, be):
    row = edge_index[0]
    col = edge_index[1]
    pad = E_PAD - E
    rowp = jnp.concatenate([row, jnp.zeros((pad,), jnp.int32)]).reshape(NW, CPT, CHUNK)
    colp = jnp.concatenate([col, jnp.full((pad,), N, jnp.int32)]).reshape(NW, CPT, CHUNK)
    degp = _sc_deg(colp)
    zer = jnp.zeros((N_PAD, D), jnp.float32)
    xb, g, acc, dis = _tc_pre(x, degp.T, gamma.reshape(1, D), beta.reshape(1, D),
                              W0, b0.reshape(1, D), Wt[0])
    for k in range(1, K + 1):
        s = _sc_hop(g, rowp, colp, zer)
        g, acc = _tc_hop(s, dis, Wt[k], acc)
    a, cl = _tc_post(xb, acc, bt.reshape(1, D), W1[:D], W1[D:], b1.reshape(1, 128),
                     W2, b2.reshape(1, 64), Wc, bc.reshape(1, 64),
                     We, be.reshape(1, 16))
    return (a, cl)


# speedup vs baseline: 6.9414x; 1.1314x over previous
import functools

import jax
import jax.numpy as jnp
from jax import lax
from jax.experimental import pallas as pl
from jax.experimental.pallas import tpu as pltpu
from jax.experimental.pallas import tpu_sc as plsc

N = 10000
D = 128
E = 640000
K = 6

NW = 32
NS = 16
CHUNK = 128
IB = 32
OUTER = 5
CPT = IB * OUTER
EPT = CPT * CHUNK
E_PAD = NW * EPT
N_PAD = 10016
RPT = 624
ZREM = N_PAD - NS * RPT
OREM = N - NS * RPT

_mesh = plsc.VectorSubcoreMesh(core_axis_name="c", subcore_axis_name="s")


def _sc_deg_body(col_hbm, out_hbm, colv, degv, sem):
    cid = lax.axis_index("c")
    sid = lax.axis_index("s")
    wid = cid * NS + sid
    pltpu.async_copy(col_hbm.at[wid], colv, sem).wait()
    zeros16 = jnp.zeros((16,), jnp.float32)

    def zbody(i, carry):
        degv[pl.ds(i * 16, 16)] = zeros16
        return carry

    lax.fori_loop(0, N_PAD // 16, zbody, 0)
    ones16 = jnp.ones((16,), jnp.float32)

    def ebody(j, carry):
        for c in range(CHUNK // 16):
            idx = colv[j, pl.ds(c * 16, 16)]
            plsc.addupdate_scatter(degv, [idx], ones16)
        return carry

    lax.fori_loop(0, CPT, ebody, 0)
    pltpu.sync_copy(degv, out_hbm.at[wid])


_sc_deg = pl.kernel(
    _sc_deg_body,
    out_type=jax.ShapeDtypeStruct((NW, N_PAD), jnp.float32),
    mesh=_mesh,
    compiler_params=pltpu.CompilerParams(needs_layout_passes=False),
    scratch_types=[
        pltpu.VMEM((CPT, CHUNK), jnp.int32),
        pltpu.VMEM((N_PAD,), jnp.float32),
        pltpu.SemaphoreType.DMA,
    ],
)


def _sc_hop_body(g_hbm, row_hbm, col_hbm, zer_hbm, out_hbm,
                 rowb, colb, rows0, rows1, acc, sem0, sem1):
    cid = lax.axis_index("c")
    sid = lax.axis_index("s")
    wid = cid * NS + sid

    def _wait(buf, sem):
        pltpu.make_async_copy(zer_hbm.at[pl.ds(0, CHUNK)], buf, sem).wait()
    pltpu.sync_copy(zer_hbm.at[pl.ds(sid * RPT, RPT)], acc.at[pl.ds(sid * RPT, RPT)])

    @pl.when(sid == 0)
    def _():
        pltpu.sync_copy(zer_hbm.at[pl.ds(NS * RPT, ZREM)], acc.at[pl.ds(NS * RPT, ZREM)])

    plsc.subcore_barrier()

    def obody(o, carry):
        pltpu.sync_copy(row_hbm.at[wid, pl.ds(o * IB, IB)], rowb)
        pltpu.sync_copy(col_hbm.at[wid, pl.ds(o * IB, IB)], colb)
        pltpu.async_copy(g_hbm.at[rowb.at[0]], rows0, sem0)

        def ibody(p, c2):
            pltpu.async_copy(g_hbm.at[rowb.at[2 * p + 1]], rows1, sem1)
            _wait(rows0, sem0)
            pltpu.sync_copy(rows0, acc.at[colb.at[2 * p]], add=True)
            pltpu.async_copy(g_hbm.at[rowb.at[2 * p + 2]], rows0, sem0)
            _wait(rows1, sem1)
            pltpu.sync_copy(rows1, acc.at[colb.at[2 * p + 1]], add=True)
            return c2

        lax.fori_loop(0, IB // 2 - 1, ibody, 0)
        pltpu.async_copy(g_hbm.at[rowb.at[IB - 1]], rows1, sem1)
        _wait(rows0, sem0)
        pltpu.sync_copy(rows0, acc.at[colb.at[IB - 2]], add=True)
        _wait(rows1, sem1)
        pltpu.sync_copy(rows1, acc.at[colb.at[IB - 1]], add=True)
        return carry

    lax.fori_loop(0, OUTER, obody, 0)
    plsc.subcore_barrier()
    pltpu.sync_copy(acc.at[pl.ds(sid * RPT, RPT)], out_hbm.at[cid, pl.ds(sid * RPT, RPT)])

    @pl.when(sid == 0)
    def _():
        pltpu.sync_copy(acc.at[pl.ds(NS * RPT, OREM)], out_hbm.at[cid, pl.ds(NS * RPT, OREM)])


_sc_hop = pl.kernel(
    _sc_hop_body,
    out_type=jax.ShapeDtypeStruct((2, N, D), jnp.float32),
    mesh=_mesh,
    compiler_params=pltpu.CompilerParams(needs_layout_passes=False),
    scratch_types=[
        pltpu.VMEM((IB, CHUNK), jnp.int32),
        pltpu.VMEM((IB, CHUNK), jnp.int32),
        pltpu.VMEM((CHUNK, D), jnp.float32),
        pltpu.VMEM((CHUNK, D), jnp.float32),
        pltpu.VMEM_SHARED((N_PAD, D), jnp.float32),
        pltpu.SemaphoreType.DMA,
        pltpu.SemaphoreType.DMA,
    ],
)


def _gelu(x):
    return x * 0.5 * (1.0 + lax.erf(x * 0.7071067811865476))


def _tc_pre_body(x_ref, degp_ref, gamma_ref, beta_ref, w0_ref, b0_ref, wt0_ref,
                 xb_ref, g_ref, acc_ref, dis_ref):
    x = x_ref[...]
    mu = jnp.mean(x, axis=0, keepdims=True)
    xc = x - mu
    var = jnp.mean(xc * xc, axis=0, keepdims=True)
    xb = xc * lax.rsqrt(var + 1e-5) * gamma_ref[...] + beta_ref[...]
    xb_ref[...] = xb
    h = _gelu(jnp.dot(xb, w0_ref[...], preferred_element_type=jnp.float32) + b0_ref[...])
    deg = jnp.sum(degp_ref[...], axis=1, keepdims=True)[:N]
    dis = jnp.where(deg > 0, lax.rsqrt(jnp.maximum(deg, 1e-12)), 0.0)
    dis_ref[...] = dis
    g_ref[...] = dis * h
    acc_ref[...] = jnp.dot(h, wt0_ref[...], preferred_element_type=jnp.float32)


_tc_pre = pl.pallas_call(
    _tc_pre_body,
    out_shape=(
        jax.ShapeDtypeStruct((N, D), jnp.float32),
        jax.ShapeDtypeStruct((N, D), jnp.float32),
        jax.ShapeDtypeStruct((N, D), jnp.float32),
        jax.ShapeDtypeStruct((N, 1), jnp.float32),
    ),
)


def _tc_hop_body(s_ref, dis_ref, wt_ref, acc_in_ref, g_ref, acc_out_ref):
    s = s_ref[0] + s_ref[1]
    dis = dis_ref[...]
    cur = dis * s
    g_ref[...] = dis * cur
    acc_out_ref[...] = acc_in_ref[...] + jnp.dot(
        cur, wt_ref[...], preferred_element_type=jnp.float32)


_tc_hop = pl.pallas_call(
    _tc_hop_body,
    out_shape=(
        jax.ShapeDtypeStruct((N, D), jnp.float32),
        jax.ShapeDtypeStruct((N, D), jnp.float32),
    ),
)


def _tc_post_body(xb_ref, acc_ref, bt_ref, w1a_ref, w1b_ref, b1_ref,
                  w2_ref, b2_ref, wc_ref, bc_ref, we_ref, be_ref,
                  a_ref, cl_ref):
    h = _gelu(acc_ref[...] + bt_ref[...])
    a1 = _gelu(jnp.dot(xb_ref[...], w1a_ref[...], preferred_element_type=jnp.float32)
               + jnp.dot(h, w1b_ref[...], preferred_element_type=jnp.float32)
               + b1_ref[...])
    a2 = _gelu(jnp.dot(a1, w2_ref[...], preferred_element_type=jnp.float32) + b2_ref[...])
    a_ref[...] = a2
    cl = _gelu(jnp.dot(a2, wc_ref[...], preferred_element_type=jnp.float32) + bc_ref[...])
    logits = jnp.dot(cl, we_ref[...], preferred_element_type=jnp.float32) + be_ref[...]
    m = jnp.max(logits, axis=1, keepdims=True)
    e = jnp.exp(logits - m)
    cl_ref[...] = e / jnp.sum(e, axis=1, keepdims=True)


_tc_post = pl.pallas_call(
    _tc_post_body,
    out_shape=(
        jax.ShapeDtypeStruct((N, 64), jnp.float32),
        jax.ShapeDtypeStruct((N, 16), jnp.float32),
    ),
)


def kernel(x, edge_index, gamma, beta, W0, b0, Wt, bt, W1, b1, W2, b2, Wc, bc, We, be):
    row = edge_index[0]
    col = edge_index[1]
    pad = E_PAD - E
    rowp = jnp.concatenate([row, jnp.zeros((pad,), jnp.int32)]).reshape(NW, CPT, CHUNK)
    colp = jnp.concatenate([col, jnp.full((pad,), N, jnp.int32)]).reshape(NW, CPT, CHUNK)
    degp = _sc_deg(colp)
    zer = jnp.zeros((N_PAD, D), jnp.float32)
    xb, g, acc, dis = _tc_pre(x, degp.T, gamma.reshape(1, D), beta.reshape(1, D),
                              W0, b0.reshape(1, D), Wt[0])
    for k in range(1, K + 1):
        s = _sc_hop(g, rowp, colp, zer)
        g, acc = _tc_hop(s, dis, Wt[k], acc)
    a, cl = _tc_post(xb, acc, bt.reshape(1, D), W1[:D], W1[D:], b1.reshape(1, 128),
                     W2, b2.reshape(1, 64), Wc, bc.reshape(1, 64),
                     We, be.reshape(1, 16))
    return (a, cl)
